# baseline (speedup 1.0000x reference)
import jax
import jax.numpy as jnp
from jax import lax
from jax.experimental import pallas as pl
from jax.experimental.pallas import tpu as pltpu
from jax.experimental.pallas import tpu_sc as plsc

N = 10000
NPAD = 10240
E = 320000
D = 128
G = 16
NC = 2
NS = 16
NW = NC * NS
C = 128
CPW = 80
SEG = 40
NSEG = CPW // SEG
ROWS = NW * CPW
RPAD = ROWS + 8
EPAD = ROWS * C
RPT = NPAD // NS

_f32 = jnp.float32
_MESH = plsc.VectorSubcoreMesh(core_axis_name="c", subcore_axis_name="s",
                               num_cores=NC, num_subcores=NS)


def _deg_body(row_hbm, w_hbm, deg_out, widx, wval, zbuf, shared_deg):
    c = lax.axis_index("c")
    s = lax.axis_index("s")
    wid = s * NC + c

    def zb(i, _):
        zbuf[pl.ds(i * 16, 16)] = jnp.zeros((16,), _f32)
        return 0
    lax.fori_loop(0, RPT // 16, zb, 0)
    pltpu.sync_copy(zbuf, shared_deg.at[pl.ds(s * RPT, RPT)])

    pltpu.sync_copy(row_hbm.at[pl.ds(wid * CPW, CPW)], widx)
    pltpu.sync_copy(w_hbm.at[pl.ds(wid * CPW, CPW)], wval)

    def ab(j, _):
        for k in range(C // 16):
            wval[j, pl.ds(k * 16, 16)] = jnp.abs(wval[j, pl.ds(k * 16, 16)])
        return 0
    lax.fori_loop(0, CPW, ab, 0)

    plsc.subcore_barrier()

    def sc_add(j, _):
        pltpu.sync_copy(wval.at[j], shared_deg.at[widx.at[j]], add=True)
        return 0
    lax.fori_loop(0, CPW, sc_add, 0)

    plsc.subcore_barrier()
    pltpu.sync_copy(shared_deg.at[pl.ds(s * RPT, RPT)],
                    deg_out.at[c, pl.ds(s * RPT, RPT)])


_deg_call = pl.kernel(
    _deg_body,
    out_type=jax.ShapeDtypeStruct((NC, NPAD), _f32),
    mesh=_MESH,
    scratch_types=[
        pltpu.VMEM((CPW, C), jnp.int32),
        pltpu.VMEM((CPW, C), _f32),
        pltpu.VMEM((RPT,), _f32),
        pltpu.VMEM_SHARED((NPAD,), _f32),
    ],
)


def _spmm_body(y_hbm, row_hbm, col_hbm, z_hbm, out_hbm, ridx, cidx, v0, v1,
               gsem, ssem, shared_out):
    c = lax.axis_index("c")
    s = lax.axis_index("s")
    wid = s * NC + c

    for t in range(RPT // C):
        pltpu.async_copy(z_hbm, shared_out.at[pl.ds(s * RPT + t * C, C)], ssem)
    for t in range(RPT // C):
        pltpu.make_async_copy(z_hbm, shared_out.at[pl.ds(s * RPT, C)], ssem).wait()

    def gather(jj, buf):
        pltpu.async_copy(y_hbm.at[ridx.at[jj]], buf, gsem)

    def gwait(buf):
        pltpu.make_async_copy(y_hbm.at[ridx.at[0]], buf, gsem).wait()

    def sscat(jj, buf):
        pltpu.async_copy(buf, shared_out.at[cidx.at[jj]], ssem, add=True)

    def swait(buf):
        pltpu.make_async_copy(y_hbm.at[ridx.at[0]], buf, ssem).wait()

    plsc.subcore_barrier()

    def seg_body(g, _):
        base = wid * CPW + g * SEG
        pltpu.sync_copy(row_hbm.at[pl.ds(base, SEG)], ridx)
        pltpu.sync_copy(col_hbm.at[pl.ds(base, SEG)], cidx)

        gather(0, v0)
        gather(1, v1)
        gwait(v0)
        sscat(0, v0)

        def ch(i, _):
            jo = 2 * i
            swait(v0)
            gather(jo + 2, v0)
            gwait(v1)
            sscat(jo + 1, v1)
            swait(v1)
            gather(jo + 3, v1)
            gwait(v0)
            sscat(jo + 2, v0)
            return 0
        lax.fori_loop(0, (SEG - 2) // 2, ch, 0)

        swait(v0)
        gwait(v1)
        sscat(SEG - 1, v1)
        swait(v1)
        return 0
    lax.fori_loop(0, NSEG, seg_body, 0)

    plsc.subcore_barrier()
    for t in range(RPT // C):
        pltpu.async_copy(shared_out.at[pl.ds(s * RPT + t * C, C)],
                         out_hbm.at[c, pl.ds(s * RPT + t * C, C)], ssem)
    for t in range(RPT // C):
        pltpu.make_async_copy(shared_out.at[pl.ds(s * RPT, C)],
                              out_hbm.at[c, pl.ds(s * RPT, C)], ssem).wait()


_spmm_call = pl.kernel(
    _spmm_body,
    out_type=jax.ShapeDtypeStruct((NC, NPAD, D), _f32),
    mesh=_MESH,
    scratch_types=[
        pltpu.VMEM((SEG, C), jnp.int32),
        pltpu.VMEM((SEG, C), jnp.int32),
        pltpu.VMEM((C, D), _f32),
        pltpu.VMEM((C, D), _f32),
        pltpu.SemaphoreType.DMA,
        pltpu.SemaphoreType.DMA,
        pltpu.VMEM_SHARED((NPAD, D), _f32),
    ],
)


def _dis_of(degt):
    deg = degt[:, 0] + degt[:, 1]
    return jnp.where(deg > 0, lax.rsqrt(jnp.where(deg > 0, deg, 1.0)), 0.0)


def _k1_body(x_ref, wt_ref, degt_ref, y_ref):
    dis = _dis_of(degt_ref[...])
    z = jnp.dot(x_ref[...], wt_ref[...], preferred_element_type=_f32)
    y_ref[...] = dis[:, None] * z


def _k2_body(acc_ref, degt_ref, wt_ref, b_ref, y_ref):
    dis = _dis_of(degt_ref[...])
    a = acc_ref[0] + acc_ref[1]
    h = dis[:, None] * a + b_ref[...][None, :]
    h = jnp.where(h >= 0, h, 0.01 * h)
    z = jnp.dot(h, wt_ref[...], preferred_element_type=_f32)
    y_ref[...] = dis[:, None] * z


def _k3_body(acc_ref, degt_ref, b_ref, batch_ref, wmt_ref, bm_ref, out_ref):
    dis = _dis_of(degt_ref[...])
    a = acc_ref[0] + acc_ref[1]
    h = dis[:, None] * a + b_ref[...][None, :]
    h = jnp.where(h >= 0, h, 0.01 * h)
    gids = lax.broadcasted_iota(jnp.int32, (G, NPAD), 0)
    P = (batch_ref[...] == gids).astype(_f32)
    sums = jnp.dot(P, h, preferred_element_type=_f32)
    counts = jnp.sum(P, axis=1)
    pooled = sums / jnp.maximum(counts, 1.0)[:, None]
    out_ref[...] = (jnp.dot(pooled, wmt_ref[...], preferred_element_type=_f32)
                    + bm_ref[...][None, :])


_BLK = 1024

_k1_call = pl.pallas_call(
    _k1_body,
    grid=(NPAD // _BLK,),
    in_specs=[
        pl.BlockSpec((_BLK, D), lambda i: (i, 0)),
        pl.BlockSpec((D, D), lambda i: (0, 0)),
        pl.BlockSpec((_BLK, 2), lambda i: (i, 0)),
    ],
    out_specs=pl.BlockSpec((_BLK, D), lambda i: (i, 0)),
    out_shape=jax.ShapeDtypeStruct((NPAD, D), _f32),
)

_k2_call = pl.pallas_call(
    _k2_body,
    grid=(NPAD // _BLK,),
    in_specs=[
        pl.BlockSpec((NC, _BLK, D), lambda i: (0, i, 0)),
        pl.BlockSpec((_BLK, 2), lambda i: (i, 0)),
        pl.BlockSpec((D, D), lambda i: (0, 0)),
        pl.BlockSpec((D,), lambda i: (0,)),
    ],
    out_specs=pl.BlockSpec((_BLK, D), lambda i: (i, 0)),
    out_shape=jax.ShapeDtypeStruct((NPAD, D), _f32),
)

_k3_call = pl.pallas_call(
    _k3_body,
    out_shape=jax.ShapeDtypeStruct((G, 2), _f32),
)


def kernel(x, edge_index, edge_weights, batch, W1, b1, W2, b2, Wm, bm):
    r = edge_index[0].astype(jnp.int32)
    co = edge_index[1].astype(jnp.int32)
    w = edge_weights.astype(_f32)
    pad_n = EPAD - E
    tail_n = (RPAD - ROWS) * C
    pidx = N + (lax.iota(jnp.int32, pad_n) % (NPAD - N))
    tail = jnp.zeros((tail_n,), jnp.int32)
    rp = jnp.concatenate([r, pidx, tail]).reshape(RPAD, C)
    cp = jnp.concatenate([co, pidx, tail]).reshape(RPAD, C)
    wp = jnp.concatenate([w, jnp.zeros((pad_n + tail_n,), _f32)]).reshape(RPAD, C)
    xp = jnp.zeros((NPAD, D), _f32).at[:N].set(x)
    batch2 = jnp.full((1, NPAD), G, jnp.int32).at[0, :N].set(batch.astype(jnp.int32))

    deg_parts = _deg_call(rp, wp)
    degt = deg_parts.T

    y1 = _k1_call(xp, W1.T, degt)
    zrow = jnp.zeros((C, D), _f32)
    acc1 = _spmm_call(y1, rp, cp, zrow)
    y2 = _k2_call(acc1, degt, W2.T, b1)
    acc2 = _spmm_call(y2, rp, cp, zrow)
    return _k3_call(acc2, degt, b2, batch2, Wm.T, bm)


# speedup vs baseline: 1.1132x; 1.1132x over previous
import jax
import jax.numpy as jnp
from jax import lax
from jax.experimental import pallas as pl
from jax.experimental.pallas import tpu as pltpu
from jax.experimental.pallas import tpu_sc as plsc

N = 10000
NPAD = 10240
E = 320000
D = 128
G = 16
NC = 2
NS = 16
NW = NC * NS
C = 128
CPW = 80
SEG = 40
NSEG = CPW // SEG
ROWS = NW * CPW
RPAD = ROWS + 8
EPAD = ROWS * C
RPT = NPAD // NS

_f32 = jnp.float32
_MESH = plsc.VectorSubcoreMesh(core_axis_name="c", subcore_axis_name="s",
                               num_cores=NC, num_subcores=NS)


def _deg_body(row_hbm, w_hbm, deg_out, widx, wval, zbuf, dsem, shared_deg):
    c = lax.axis_index("c")
    s = lax.axis_index("s")
    wid = s * NC + c

    def zb(i, _):
        zbuf[pl.ds(i * 16, 16)] = jnp.zeros((16,), _f32)
        return 0
    lax.fori_loop(0, RPT // 16, zb, 0)
    pltpu.sync_copy(zbuf, shared_deg.at[pl.ds(s * RPT, RPT)])

    pltpu.sync_copy(row_hbm.at[pl.ds(wid * CPW, CPW)], widx)
    pltpu.sync_copy(w_hbm.at[pl.ds(wid * CPW, CPW)], wval)

    def ab(j, _):
        for k in range(C // 16):
            wval[j, pl.ds(k * 16, 16)] = jnp.abs(wval[j, pl.ds(k * 16, 16)])
        return 0
    lax.fori_loop(0, CPW, ab, 0)

    plsc.subcore_barrier()

    def sc_add(g, _):
        for u in range(8):
            pltpu.async_copy(wval.at[8 * g + u],
                             shared_deg.at[widx.at[8 * g + u]], dsem, add=True)
        for u in range(8):
            pltpu.make_async_copy(w_hbm.at[0], wval.at[0], dsem).wait()
        return 0
    lax.fori_loop(0, CPW // 8, sc_add, 0)

    plsc.subcore_barrier()
    pltpu.sync_copy(shared_deg.at[pl.ds(s * RPT, RPT)],
                    deg_out.at[c, pl.ds(s * RPT, RPT)])


_deg_call = pl.kernel(
    _deg_body,
    out_type=jax.ShapeDtypeStruct((NC, NPAD), _f32),
    mesh=_MESH,
    scratch_types=[
        pltpu.VMEM((CPW, C), jnp.int32),
        pltpu.VMEM((CPW, C), _f32),
        pltpu.VMEM((RPT,), _f32),
        pltpu.SemaphoreType.DMA,
        pltpu.VMEM_SHARED((NPAD,), _f32),
    ],
)


def _spmm_body(y_hbm, row_hbm, col_hbm, out_hbm, ridx, cidx, v0, v1,
               gsem, ssem, shared_out):
    c = lax.axis_index("c")
    s = lax.axis_index("s")
    wid = s * NC + c

    def zb(i, _):
        for k in range(D // 16):
            v0[i, pl.ds(k * 16, 16)] = jnp.zeros((16,), _f32)
        return 0
    lax.fori_loop(0, C, zb, 0)

    for t in range(RPT // C):
        pltpu.async_copy(v0, shared_out.at[pl.ds(s * RPT + t * C, C)], ssem)
    for t in range(RPT // C):
        pltpu.make_async_copy(v0, shared_out.at[pl.ds(s * RPT, C)], ssem).wait()

    def gather(jj, buf):
        pltpu.async_copy(y_hbm.at[ridx.at[jj]], buf, gsem)

    def gwait(buf):
        pltpu.make_async_copy(y_hbm.at[ridx.at[0]], buf, gsem).wait()

    def sscat(jj, buf):
        pltpu.async_copy(buf, shared_out.at[cidx.at[jj]], ssem, add=True)

    def swait(buf):
        pltpu.make_async_copy(y_hbm.at[ridx.at[0]], buf, ssem).wait()

    plsc.subcore_barrier()

    def seg_body(g, _):
        base = wid * CPW + g * SEG
        pltpu.sync_copy(row_hbm.at[pl.ds(base, SEG)], ridx)
        pltpu.sync_copy(col_hbm.at[pl.ds(base, SEG)], cidx)

        gather(0, v0)
        gather(1, v1)
        gwait(v0)
        sscat(0, v0)

        bufs = (v0, v1)

        def step(k, pu):
            swait(bufs[1 - pu])
            gather(k + 1, bufs[1 - pu])
            gwait(bufs[pu])
            sscat(k, bufs[pu])

        def ch(i, _):
            jo = 4 * i
            for u in range(4):
                step(jo + 1 + u, (1 + u) % 2)
            return 0
        lax.fori_loop(0, (SEG - 4) // 4, ch, 0)
        step(SEG - 3, (SEG - 3) % 2)
        step(SEG - 2, (SEG - 2) % 2)

        swait(v0)
        gwait(v1)
        sscat(SEG - 1, v1)
        swait(v1)
        return 0
    lax.fori_loop(0, NSEG, seg_body, 0)

    plsc.subcore_barrier()
    for t in range(RPT // C):
        pltpu.async_copy(shared_out.at[pl.ds(s * RPT + t * C, C)],
                         out_hbm.at[c, pl.ds(s * RPT + t * C, C)], ssem)
    for t in range(RPT // C):
        pltpu.make_async_copy(shared_out.at[pl.ds(s * RPT, C)],
                              out_hbm.at[c, pl.ds(s * RPT, C)], ssem).wait()


_spmm_call = pl.kernel(
    _spmm_body,
    out_type=jax.ShapeDtypeStruct((NC, NPAD, D), _f32),
    mesh=_MESH,
    scratch_types=[
        pltpu.VMEM((SEG, C), jnp.int32),
        pltpu.VMEM((SEG, C), jnp.int32),
        pltpu.VMEM((C, D), _f32),
        pltpu.VMEM((C, D), _f32),
        pltpu.SemaphoreType.DMA,
        pltpu.SemaphoreType.DMA,
        pltpu.VMEM_SHARED((NPAD, D), _f32),
    ],
)


def _dis_of(degt):
    deg = degt[:, 0] + degt[:, 1]
    return jnp.where(deg > 0, lax.rsqrt(jnp.where(deg > 0, deg, 1.0)), 0.0)


def _k1_body(x_ref, wt_ref, degt_ref, y_ref):
    dis = _dis_of(degt_ref[...])
    z = jnp.dot(x_ref[...], wt_ref[...], preferred_element_type=_f32)
    y_ref[...] = dis[:, None] * z


def _k2_body(acc_ref, degt_ref, wt_ref, b_ref, y_ref):
    dis = _dis_of(degt_ref[...])
    a = acc_ref[0] + acc_ref[1]
    h = dis[:, None] * a + b_ref[...][None, :]
    h = jnp.where(h >= 0, h, 0.01 * h)
    z = jnp.dot(h, wt_ref[...], preferred_element_type=_f32)
    y_ref[...] = dis[:, None] * z


def _k3_body(acc_ref, degt_ref, b_ref, batch_ref, wmt_ref, bm_ref, out_ref):
    dis = _dis_of(degt_ref[...])
    a = acc_ref[0] + acc_ref[1]
    h = dis[:, None] * a + b_ref[...][None, :]
    h = jnp.where(h >= 0, h, 0.01 * h)
    gids = lax.broadcasted_iota(jnp.int32, (G, NPAD), 0)
    P = (batch_ref[...] == gids).astype(_f32)
    sums = jnp.dot(P, h, preferred_element_type=_f32)
    counts = jnp.sum(P, axis=1)
    pooled = sums / jnp.maximum(counts, 1.0)[:, None]
    out_ref[...] = (jnp.dot(pooled, wmt_ref[...], preferred_element_type=_f32)
                    + bm_ref[...][None, :])


_BLK = 1024

_k1_call = pl.pallas_call(
    _k1_body,
    grid=(NPAD // _BLK,),
    in_specs=[
        pl.BlockSpec((_BLK, D), lambda i: (i, 0)),
        pl.BlockSpec((D, D), lambda i: (0, 0)),
        pl.BlockSpec((_BLK, 2), lambda i: (i, 0)),
    ],
    out_specs=pl.BlockSpec((_BLK, D), lambda i: (i, 0)),
    out_shape=jax.ShapeDtypeStruct((NPAD, D), _f32),
)

_k2_call = pl.pallas_call(
    _k2_body,
    grid=(NPAD // _BLK,),
    in_specs=[
        pl.BlockSpec((NC, _BLK, D), lambda i: (0, i, 0)),
        pl.BlockSpec((_BLK, 2), lambda i: (i, 0)),
        pl.BlockSpec((D, D), lambda i: (0, 0)),
        pl.BlockSpec((D,), lambda i: (0,)),
    ],
    out_specs=pl.BlockSpec((_BLK, D), lambda i: (i, 0)),
    out_shape=jax.ShapeDtypeStruct((NPAD, D), _f32),
)

_k3_call = pl.pallas_call(
    _k3_body,
    out_shape=jax.ShapeDtypeStruct((G, 2), _f32),
)


def kernel(x, edge_index, edge_weights, batch, W1, b1, W2, b2, Wm, bm):
    r = edge_index[0].astype(jnp.int32)
    co = edge_index[1].astype(jnp.int32)
    w = edge_weights.astype(_f32)
    pad_n = EPAD - E
    tail_n = (RPAD - ROWS) * C
    pidx = N + (lax.iota(jnp.int32, pad_n) % (NPAD - N))
    tail = jnp.zeros((tail_n,), jnp.int32)
    rp = jnp.concatenate([r, pidx, tail]).reshape(RPAD, C)
    cp = jnp.concatenate([co, pidx, tail]).reshape(RPAD, C)
    wp = jnp.concatenate([w, jnp.zeros((pad_n + tail_n,), _f32)]).reshape(RPAD, C)
    xp = jnp.zeros((NPAD, D), _f32).at[:N].set(x)
    batch2 = jnp.full((1, NPAD), G, jnp.int32).at[0, :N].set(batch.astype(jnp.int32))

    deg_parts = _deg_call(rp, wp)
    degt = deg_parts.T

    y1 = _k1_call(xp, W1.T, degt)
    acc1 = _spmm_call(y1, rp, cp)
    y2 = _k2_call(acc1, degt, W2.T, b1)
    acc2 = _spmm_call(y2, rp, cp)
    return _k3_call(acc2, degt, b2, batch2, Wm.T, bm)
